# baseline (speedup 1.0000x reference)
import functools

import jax
import jax.numpy as jnp
from jax import lax
from jax.experimental import pallas as pl
from jax.experimental.pallas import tpu as pltpu
from jax.experimental.pallas import tpu_sc as plsc

_BM = 1024
_CHUNK = 256


def _scores_body(x_ref, gw_ref, gb_ref, o_ref):
    o_ref[...] = jax.lax.dot_general(
        x_ref[...], gw_ref[...], (((1,), (1,)), ((), ()))) + gb_ref[...]


def _expert_body(x_ref, g_ref, w_ref, b_ref, o_ref):
    xb = x_ref[...]
    E = b_ref.shape[0]
    g = g_ref[...]
    acc = jax.lax.dot_general(g, b_ref[...], (((1,), (0,)), ((), ())))
    for e in range(E):
        ye = jax.lax.dot_general(
            xb, w_ref[e], (((1,), (1,)), ((), ())))
        acc = acc + g[:, e:e + 1] * ye
    o_ref[...] = acc


def _make_router(S, E):
    info = plsc.get_sparse_core_info()
    NC, NS = info.num_cores, info.num_subcores
    n_sub = NC * NS
    assert S % (n_sub * 16) == 0
    chunk = S // n_sub
    words = chunk * E
    mesh = plsc.VectorSubcoreMesh(core_axis_name="c", subcore_axis_name="s")

    @functools.partial(
        pl.kernel, mesh=mesh,
        compiler_params=pltpu.CompilerParams(needs_layout_passes=False),
        out_type=jax.ShapeDtypeStruct((S * E,), jnp.float32),
        scratch_types=[
            pltpu.VMEM((words,), jnp.float32),
            pltpu.VMEM((words,), jnp.float32),
        ],
    )
    def router(s_hbm, g_hbm, s_v, g_v):
        wid = lax.axis_index("s") * NC + lax.axis_index("c")
        base = wid * words
        pltpu.sync_copy(s_hbm.at[pl.ds(base, words)], s_v)
        zero = jnp.zeros((16,), jnp.float32)
        for q in range(words // 16):
            g_v[pl.ds(q * 16, 16)] = zero
        lane = lax.iota(jnp.int32, 16)
        for j in range(chunk // 16):
            sbase = (j * 16 + lane) * E
            sv = [plsc.load_gather(s_v, [sbase + e]) for e in range(E)]
            m1 = sv[0]
            for e in range(1, E):
                m1 = jnp.maximum(m1, sv[e])
            i1 = jnp.full((16,), E, jnp.int32)
            for e in range(E - 1, -1, -1):
                i1 = jnp.where(sv[e] == m1, e, i1)
            m2 = jnp.full((16,), -1e30, jnp.float32)
            for e in range(E):
                m2 = jnp.maximum(m2, jnp.where(i1 == e, -1e30, sv[e]))
            i2 = jnp.full((16,), E, jnp.int32)
            for e in range(E - 1, -1, -1):
                i2 = jnp.where((sv[e] == m2) & (i1 != e), e, i2)
            e2 = jnp.exp(m2 - m1)
            den = 1.0 + e2
            w1 = 1.0 / den
            w2 = e2 / den
            plsc.store_scatter(g_v, [sbase + i1], w1)
            plsc.store_scatter(g_v, [sbase + i2], w2)
        pltpu.sync_copy(g_v, g_hbm.at[pl.ds(base, words)])

    return router


def kernel(x, expert_w, expert_b, gate_w, gate_b):
    B, S, D = x.shape
    E, O, _ = expert_w.shape
    total = B * S
    x2 = x.reshape(total, D)
    gb2 = gate_b.reshape(1, E)
    scores = pl.pallas_call(
        _scores_body,
        grid=(total // _BM,),
        in_specs=[
            pl.BlockSpec((_BM, D), lambda i: (i, 0)),
            pl.BlockSpec((E, D), lambda i: (0, 0)),
            pl.BlockSpec((1, E), lambda i: (0, 0)),
        ],
        out_specs=pl.BlockSpec((_BM, E), lambda i: (i, 0)),
        out_shape=jax.ShapeDtypeStruct((total, E), jnp.float32),
    )(x2, gate_w, gb2)
    g = _make_router(total, E)(scores.reshape(total * E)).reshape(total, E)
    out = pl.pallas_call(
        _expert_body,
        grid=(total // _BM,),
        in_specs=[
            pl.BlockSpec((_BM, D), lambda i: (i, 0)),
            pl.BlockSpec((_BM, E), lambda i: (i, 0)),
            pl.BlockSpec((E, O, D), lambda i: (0, 0, 0)),
            pl.BlockSpec((E, O), lambda i: (0, 0)),
        ],
        out_specs=pl.BlockSpec((_BM, O), lambda i: (i, 0)),
        out_shape=jax.ShapeDtypeStruct((total, O), jnp.float32),
    )(x2, g, expert_w, expert_b)
    return out.reshape(B, S, O)
